# baseline (speedup 1.0000x reference)
import functools

import jax
import jax.numpy as jnp
from jax import lax
from jax.experimental import pallas as pl
from jax.experimental.pallas import tpu as pltpu
from jax.experimental.pallas import tpu_sc as plsc

_B, _L = 4096, 50
_De, _H = 256, 1024
_N = _B * _L
_R = 1024
_BPL = _B // _R
_CHUNK = 128
_SEG = 5
_LSEG = _L // _SEG
_NSEG = _N // _SEG
_EPS = 1e-12
_NPOS, _NLINK, _NPRIOR = 512, 64, 64
_NCAT = _NPOS + _NLINK + _NPRIOR


def _sc_gather(table, idx):
    info = plsc.get_sparse_core_info()
    nw = info.num_cores * info.num_subcores
    n = idx.shape[0]
    d = table.shape[1]
    b_per_w = n // nw
    n_chunks = b_per_w // _CHUNK
    mesh = plsc.VectorSubcoreMesh(core_axis_name="c", subcore_axis_name="s")

    @functools.partial(
        pl.kernel,
        out_type=jax.ShapeDtypeStruct((n, d), table.dtype),
        mesh=mesh,
        compiler_params=pltpu.CompilerParams(use_tc_tiling_on_sc=True),
        scratch_types=[
            pltpu.VMEM((b_per_w,), jnp.int32),
            pltpu.VMEM((_CHUNK, d), table.dtype),
            pltpu.SemaphoreType.DMA,
        ],
    )
    def k(table_hbm, idx_hbm, out_hbm, idx_v, rows_v, sem):
        wid = lax.axis_index("s") * info.num_cores + lax.axis_index("c")
        base = wid * b_per_w
        pltpu.sync_copy(idx_hbm.at[pl.ds(base, b_per_w)], idx_v)

        def body(c, _):
            off = c * _CHUNK
            pltpu.async_copy(
                table_hbm.at[idx_v.at[pl.ds(off, _CHUNK)]], rows_v, sem
            ).wait()
            pltpu.sync_copy(rows_v, out_hbm.at[pl.ds(base + off, _CHUNK)])
            return 0

        lax.fori_loop(0, n_chunks, body, 0)

    return k(table, idx)


def _tc_body(*refs):
    if len(refs) == 12:
        (pos_ref, link_ref, prior_ref, tt_ref, ent_ref, w_ref, cat_ref,
         type_ref, lnw_ref, lnb_ref, _prev_ref, out_ref) = refs
    else:
        (pos_ref, link_ref, prior_ref, tt_ref, ent_ref, w_ref, cat_ref,
         type_ref, lnw_ref, lnb_ref, out_ref) = refs
    acc = jnp.dot(ent_ref[...], w_ref[...], preferred_element_type=jnp.float32)

    pos = pos_ref[0, 0, :]
    link = link_ref[0, 0, :]
    prior = prior_ref[0, 0, :]
    tt = tt_ref[0, 0, :]

    colp = lax.broadcasted_iota(jnp.int32, (_R, _NPOS), 1)
    ohp = (colp == pos[:, None]).astype(jnp.float32)
    acc = acc + jnp.dot(ohp, cat_ref[0:_NPOS, :],
                        preferred_element_type=jnp.float32)
    cols = lax.broadcasted_iota(jnp.int32, (_R, _NLINK), 1)
    ohl = (cols == link[:, None]).astype(jnp.float32)
    acc = acc + jnp.dot(ohl, cat_ref[_NPOS:_NPOS + _NLINK, :],
                        preferred_element_type=jnp.float32)
    ohq = (cols == prior[:, None]).astype(jnp.float32)
    acc = acc + jnp.dot(ohq, cat_ref[_NPOS + _NLINK:_NCAT, :],
                        preferred_element_type=jnp.float32)

    t0 = type_ref[0, :][None, :]
    t1 = type_ref[1, :][None, :]
    acc = acc + t0 + tt.astype(jnp.float32)[:, None] * (t1 - t0)

    u = jnp.mean(acc, axis=1, keepdims=True)
    dlt = acc - u
    s = jnp.mean(dlt * dlt, axis=1, keepdims=True)
    y = dlt * lax.rsqrt(s + _EPS)
    out_ref[0, :, :] = y * lnw_ref[...] + lnb_ref[...]


def _tc_segment(seg, ent_seg, w, cat, type_pad, id_segs, lnw, lnb, prev):
    nb = _NSEG // _R
    l0 = seg * _LSEG
    idspec = pl.BlockSpec((1, 1, _R), lambda i: (i, 0, 0))
    in_specs = [
        idspec, idspec, idspec, idspec,
        pl.BlockSpec((_R, _De), lambda i: (i, 0)),
        pl.BlockSpec((_De, _H), lambda i: (0, 0)),
        pl.BlockSpec((_NCAT, _H), lambda i: (0, 0)),
        pl.BlockSpec((8, _H), lambda i: (0, 0)),
        pl.BlockSpec((1, _H), lambda i: (0, 0)),
        pl.BlockSpec((1, _H), lambda i: (0, 0)),
    ]
    args = [*id_segs, ent_seg, w, cat, type_pad, lnw, lnb]
    aliases = {}
    if prev is not None:
        in_specs.append(pl.BlockSpec(memory_space=pl.ANY))
        args.append(prev)
        aliases = {10: 0}
    return pl.pallas_call(
        _tc_body,
        grid=(nb,),
        in_specs=in_specs,
        out_specs=pl.BlockSpec(
            (1, _R, _H), lambda i: (l0 + i // _BPL, i % _BPL, 0)),
        out_shape=jax.ShapeDtypeStruct((_L, _B, _H), jnp.float32),
        input_output_aliases=aliases,
    )(*args)


def kernel(entity_ids, position_ids, token_type_ids, link_prob_ids,
           prior_prob_ids, entity_table, pos_table, type_table, link_table,
           prior_table, W, ln_w, ln_b):
    idx_lm = entity_ids.T.reshape(-1)
    ent_segs = [
        _sc_gather(entity_table, idx_lm[s * _NSEG:(s + 1) * _NSEG])
        for s in range(_SEG)
    ]

    ids = [
        s.T.reshape(_N // _R, 1, _R)
        for s in (position_ids, link_prob_ids, prior_prob_ids,
                  token_type_ids)
    ]

    cat = jnp.concatenate([pos_table, link_table, prior_table], axis=0)
    type_pad = jnp.concatenate(
        [type_table, jnp.zeros((6, _H), jnp.float32)], axis=0)
    lnw = ln_w.reshape(1, _H)
    lnb = ln_b.reshape(1, _H)

    nbs = _NSEG // _R
    out = None
    for s in range(_SEG):
        id_segs = [a[s * nbs:(s + 1) * nbs] for a in ids]
        out = _tc_segment(s, ent_segs[s], W, cat, type_pad,
                          id_segs, lnw, lnb, out)
    return out.transpose(1, 0, 2)


# speedup vs baseline: 1.1189x; 1.1189x over previous
import functools

import jax
import jax.numpy as jnp
from jax import lax
from jax.experimental import pallas as pl
from jax.experimental.pallas import tpu as pltpu
from jax.experimental.pallas import tpu_sc as plsc

_B, _L = 4096, 50
_De, _H = 256, 1024
_N = _B * _L
_R = 1024
_BPL = _B // _R
_CHUNK = 128
_SEG = 5
_LSEG = _L // _SEG
_NSEG = _N // _SEG
_EPS = 1e-12
_NPOS, _NLINK, _NPRIOR = 512, 64, 64
_NCAT = _NPOS + _NLINK + _NPRIOR


def _sc_gather(table, idx):
    info = plsc.get_sparse_core_info()
    nw = info.num_cores * info.num_subcores
    n = idx.shape[0]
    d = table.shape[1]
    b_per_w = n // nw
    n_chunks = b_per_w // _CHUNK
    mesh = plsc.VectorSubcoreMesh(core_axis_name="c", subcore_axis_name="s")

    @functools.partial(
        pl.kernel,
        out_type=jax.ShapeDtypeStruct((n, d), table.dtype),
        mesh=mesh,
        compiler_params=pltpu.CompilerParams(use_tc_tiling_on_sc=True),
        scratch_types=[
            pltpu.VMEM((b_per_w,), jnp.int32),
            pltpu.VMEM((_CHUNK, d), table.dtype),
            pltpu.SemaphoreType.DMA,
        ],
    )
    def k(table_hbm, idx_hbm, out_hbm, idx_v, rows_v, sem):
        wid = lax.axis_index("s") * info.num_cores + lax.axis_index("c")
        base = wid * b_per_w
        pltpu.sync_copy(idx_hbm.at[pl.ds(base, b_per_w)], idx_v)

        def body(c, _):
            off = c * _CHUNK
            pltpu.async_copy(
                table_hbm.at[idx_v.at[pl.ds(off, _CHUNK)]], rows_v, sem
            ).wait()
            pltpu.sync_copy(rows_v, out_hbm.at[pl.ds(base + off, _CHUNK)])
            return 0

        lax.fori_loop(0, n_chunks, body, 0)

    return k(table, idx)


def _tc_body(*refs):
    if len(refs) == 12:
        (pos_ref, link_ref, prior_ref, tt_ref, ent_ref, w_ref, cat_ref,
         type_ref, lnw_ref, lnb_ref, _prev_ref, out_ref) = refs
    else:
        (pos_ref, link_ref, prior_ref, tt_ref, ent_ref, w_ref, cat_ref,
         type_ref, lnw_ref, lnb_ref, out_ref) = refs
    acc = jnp.dot(ent_ref[...], w_ref[...],
                  precision=lax.Precision.DEFAULT,
                  preferred_element_type=jnp.float32)

    pos = pos_ref[0, 0, :]
    link = link_ref[0, 0, :]
    prior = prior_ref[0, 0, :]
    tt = tt_ref[0, 0, :]

    col = lax.broadcasted_iota(jnp.int32, (_R, _NCAT), 1)
    oh = ((col == pos[:, None]).astype(jnp.float32)
          + (col == link[:, None] + _NPOS).astype(jnp.float32)
          + (col == prior[:, None] + (_NPOS + _NLINK)).astype(jnp.float32))
    acc = acc + jnp.dot(oh, cat_ref[...],
                        precision=lax.Precision.DEFAULT,
                        preferred_element_type=jnp.float32)

    t0 = type_ref[0, :][None, :]
    t1 = type_ref[1, :][None, :]
    acc = acc + jnp.where(tt[:, None] == 1, t1, t0)

    u = jnp.mean(acc, axis=1, keepdims=True)
    dlt = acc - u
    s = jnp.mean(dlt * dlt, axis=1, keepdims=True)
    y = dlt * lax.rsqrt(s + _EPS)
    out_ref[0, :, :] = y * lnw_ref[...] + lnb_ref[...]


def _tc_segment(seg, ent_seg, w, cat, type_pad, id_segs, lnw, lnb, prev):
    nb = _NSEG // _R
    l0 = seg * _LSEG
    idspec = pl.BlockSpec((1, 1, _R), lambda i: (i, 0, 0))
    in_specs = [
        idspec, idspec, idspec, idspec,
        pl.BlockSpec((_R, _De), lambda i: (i, 0)),
        pl.BlockSpec((_De, _H), lambda i: (0, 0)),
        pl.BlockSpec((_NCAT, _H), lambda i: (0, 0)),
        pl.BlockSpec((8, _H), lambda i: (0, 0)),
        pl.BlockSpec((1, _H), lambda i: (0, 0)),
        pl.BlockSpec((1, _H), lambda i: (0, 0)),
    ]
    args = [*id_segs, ent_seg, w, cat, type_pad, lnw, lnb]
    aliases = {}
    if prev is not None:
        in_specs.append(pl.BlockSpec(memory_space=pl.ANY))
        args.append(prev)
        aliases = {10: 0}
    return pl.pallas_call(
        _tc_body,
        grid=(nb,),
        in_specs=in_specs,
        out_specs=pl.BlockSpec(
            (1, _R, _H), lambda i: (l0 + i // _BPL, i % _BPL, 0)),
        out_shape=jax.ShapeDtypeStruct((_L, _B, _H), jnp.float32),
        input_output_aliases=aliases,
    )(*args)


def kernel(entity_ids, position_ids, token_type_ids, link_prob_ids,
           prior_prob_ids, entity_table, pos_table, type_table, link_table,
           prior_table, W, ln_w, ln_b):
    idx_lm = entity_ids.T.reshape(-1)
    ent_segs = [
        _sc_gather(entity_table, idx_lm[s * _NSEG:(s + 1) * _NSEG])
        for s in range(_SEG)
    ]

    ids = [
        s.T.reshape(_N // _R, 1, _R)
        for s in (position_ids, link_prob_ids, prior_prob_ids,
                  token_type_ids)
    ]

    cat = jnp.concatenate([pos_table, link_table, prior_table], axis=0)
    type_pad = jnp.concatenate(
        [type_table, jnp.zeros((6, _H), jnp.float32)], axis=0)
    lnw = ln_w.reshape(1, _H)
    lnb = ln_b.reshape(1, _H)

    nbs = _NSEG // _R
    out = None
    for s in range(_SEG):
        id_segs = [a[s * nbs:(s + 1) * nbs] for a in ids]
        out = _tc_segment(s, ent_segs[s], W, cat, type_pad,
                          id_segs, lnw, lnb, out)
    return out.transpose(1, 0, 2)
